# baseline (speedup 1.0000x reference)
import jax
import jax.numpy as jnp
from jax.experimental import pallas as pl
from jax.experimental.pallas import tpu as pltpu

_N = 16384
_BI = 1024
_CH = 2048
_G = _N // _BI
_NCH = _N // _CH


def _centered_counts(col_ref, row_ref):
    yi = col_ref[...]
    acc = jnp.zeros((_BI, _CH), jnp.float32)
    for jc in range(_NCH):
        yj = row_ref[:, pl.ds(jc * _CH, _CH)]
        acc = acc + (yj < yi).astype(jnp.float32)
    mean_rank = (_N - 1) / 2.0
    return jnp.sum(acc, axis=1, keepdims=True) - mean_rank


def _body(yp_col_ref, yp_row_ref, yt_col_ref, yt_row_ref, out_ref, acc_ref):
    g = pl.program_id(0)

    c_p = _centered_counts(yp_col_ref, yp_row_ref)
    c_t = _centered_counts(yt_col_ref, yt_row_ref)

    s_pt = jnp.sum(c_p * c_t)
    s_pp = jnp.sum(c_p * c_p)
    s_tt = jnp.sum(c_t * c_t)
    s_p = jnp.sum(c_p)
    s_t = jnp.sum(c_t)

    @pl.when(g == 0)
    def _init():
        for k in range(8):
            acc_ref[k] = 0.0

    acc_ref[0] = acc_ref[0] + s_pt
    acc_ref[1] = acc_ref[1] + s_pp
    acc_ref[2] = acc_ref[2] + s_tt
    acc_ref[3] = acc_ref[3] + s_p
    acc_ref[4] = acc_ref[4] + s_t

    @pl.when(g == _G - 1)
    def _finish():
        n = jnp.float32(_N)
        sp = acc_ref[3]
        st = acc_ref[4]
        num = acc_ref[0] - sp * st / n
        var_p = acc_ref[1] - sp * sp / n
        var_t = acc_ref[2] - st * st / n
        den = jnp.sqrt(var_p * var_t)
        loss = 1.0 - num / (den + 1e-8)
        out_ref[:, :] = jnp.full((8, 128), loss, jnp.float32)


def kernel(y_pred, y_true):
    yp_col = y_pred.reshape(_N, 1)
    yp_row = y_pred.reshape(1, _N)
    yt_col = y_true.reshape(_N, 1)
    yt_row = y_true.reshape(1, _N)
    out = pl.pallas_call(
        _body,
        grid=(_G,),
        in_specs=[
            pl.BlockSpec((_BI, 1), lambda g: (g, 0)),
            pl.BlockSpec((1, _N), lambda g: (0, 0)),
            pl.BlockSpec((_BI, 1), lambda g: (g, 0)),
            pl.BlockSpec((1, _N), lambda g: (0, 0)),
        ],
        out_specs=pl.BlockSpec((8, 128), lambda g: (0, 0)),
        out_shape=jax.ShapeDtypeStruct((8, 128), jnp.float32),
        scratch_shapes=[pltpu.SMEM((8,), jnp.float32)],
    )(yp_col, yp_row, yt_col, yt_row)
    return out[0, 0]


# speedup vs baseline: 11.0584x; 11.0584x over previous
import jax
import jax.numpy as jnp
from jax.experimental import pallas as pl

_N = 16384
_R = 128
_C = 128
_MEAN = (_N - 1) / 2.0
_SUMSQ = float(_N) * (float(_N) ** 2 - 1.0) / 12.0


def _xor_partner(x, j):
    if j < _C:
        left = jnp.concatenate([x[:, j:], x[:, :j]], axis=1)
        right = jnp.concatenate([x[:, -j:], x[:, :-j]], axis=1)
    else:
        s = j // _C
        left = jnp.concatenate([x[s:, :], x[:s, :]], axis=0)
        right = jnp.concatenate([x[-s:, :], x[:-s, :]], axis=0)
    return left, right


def _bitonic_sort(key, pay, bit):
    for lk in range(1, 15):
        k = 1 << lk
        asc = (bit[k] == 0) if k < _N else None
        for lj in range(lk - 1, -1, -1):
            j = 1 << lj
            bitj = bit[j] != 0
            kl, kr = _xor_partner(key, j)
            p_key = jnp.where(bitj, kr, kl)
            pl_, pr = _xor_partner(pay, j)
            p_pay = jnp.where(bitj, pr, pl_)
            if asc is None:
                sel = bitj == 0
            else:
                sel = asc != bitj
            swap = (sel & (key > p_key)) | (~sel & (p_key > key))
            key = jnp.where(swap, p_key, key)
            pay = jnp.where(swap, p_pay, pay)
    return key, pay


def _body(yp_ref, yt_ref, out_ref):
    rows = jax.lax.broadcasted_iota(jnp.int32, (_R, _C), 0)
    cols = jax.lax.broadcasted_iota(jnp.int32, (_R, _C), 1)
    flat = rows * _C + cols
    bit = {1 << b: flat & (1 << b) for b in range(14)}

    _, z = _bitonic_sort(yp_ref[...], yt_ref[...], bit)
    _, u = _bitonic_sort(z, flat.astype(jnp.float32), bit)

    num = jnp.sum((flat.astype(jnp.float32) - _MEAN) * (u - _MEAN))
    loss = 1.0 - num / (jnp.float32(_SUMSQ) + 1e-8)
    out_ref[:, :] = jnp.full((8, _C), loss, jnp.float32)


def kernel(y_pred, y_true):
    yp = y_pred.reshape(_R, _C)
    yt = y_true.reshape(_R, _C)
    out = pl.pallas_call(
        _body,
        in_specs=[
            pl.BlockSpec((_R, _C), lambda: (0, 0)),
            pl.BlockSpec((_R, _C), lambda: (0, 0)),
        ],
        out_specs=pl.BlockSpec((8, _C), lambda: (0, 0)),
        out_shape=jax.ShapeDtypeStruct((8, _C), jnp.float32),
        grid=(),
    )(yp, yt)
    return out[0, 0]


# speedup vs baseline: 18.1193x; 1.6385x over previous
import jax
import jax.numpy as jnp
from jax.experimental import pallas as pl

_N = 16384
_R = 128
_C = 128
_MEAN = (_N - 1) / 2.0
_SUMSQ = float(_N) * (float(_N) ** 2 - 1.0) / 12.0


def _xor_sub(x, j):
    nb = _R // (2 * j)
    x4 = x.reshape(nb, 2, j, _C)
    return jnp.concatenate([x4[:, 1:2], x4[:, 0:1]], axis=1).reshape(_R, _C)


def _xor_lane(x, j, bitj):
    s = j // _C
    left = jnp.concatenate([x[:, s:], x[:, :s]], axis=1)
    right = jnp.concatenate([x[:, -s:], x[:, :-s]], axis=1)
    return jnp.where(bitj, right, left)


def _bitonic_sort(key, pay, rb, cb):
    for lk in range(1, 15):
        for lj in range(lk - 1, -1, -1):
            j = 1 << lj
            bj = rb[lj] if lj <= 6 else cb[lj - 7]
            if lk == 14:
                sel = bj == 0
            else:
                bk = rb[lk] if lk <= 6 else cb[lk - 7]
                sel = bj == bk
            if lj <= 6:
                p_key = _xor_sub(key, j)
                p_pay = _xor_sub(pay, j)
            else:
                bjm = bj != 0
                p_key = _xor_lane(key, j, bjm)
                p_pay = _xor_lane(pay, j, bjm)
            new_key = jnp.where(sel, jnp.minimum(key, p_key),
                                jnp.maximum(key, p_key))
            swap = new_key != key
            pay = jnp.where(swap, p_pay, pay)
            key = new_key
    return key, pay


def _body(yp_ref, yt_ref, out_ref):
    rows = jax.lax.broadcasted_iota(jnp.int32, (_R, 1), 0)
    cols = jax.lax.broadcasted_iota(jnp.int32, (1, _C), 1)
    rb = [(rows >> b) & 1 for b in range(7)]
    cb = [(cols >> b) & 1 for b in range(7)]
    flat = (cols * _C + rows).astype(jnp.float32)

    _, z = _bitonic_sort(yp_ref[...], yt_ref[...], rb, cb)
    _, u = _bitonic_sort(z, jnp.broadcast_to(flat, (_R, _C)), rb, cb)

    num = jnp.sum((flat - _MEAN) * (u - _MEAN))
    loss = 1.0 - num / (jnp.float32(_SUMSQ) + 1e-8)
    out_ref[:, :] = jnp.full((8, _C), loss, jnp.float32)


def kernel(y_pred, y_true):
    yp = y_pred.reshape(_R, _C)
    yt = y_true.reshape(_R, _C)
    out = pl.pallas_call(
        _body,
        in_specs=[
            pl.BlockSpec((_R, _C), lambda: (0, 0)),
            pl.BlockSpec((_R, _C), lambda: (0, 0)),
        ],
        out_specs=pl.BlockSpec((8, _C), lambda: (0, 0)),
        out_shape=jax.ShapeDtypeStruct((8, _C), jnp.float32),
        grid=(),
    )(yp, yt)
    return out[0, 0]


# speedup vs baseline: 18.2399x; 1.0067x over previous
import jax
import jax.numpy as jnp
from jax.experimental import pallas as pl

_N = 16384
_R = 128
_C = 128
_MEAN = (_N - 1) / 2.0
_SUMSQ = float(_N) * (float(_N) ** 2 - 1.0) / 12.0


def _xor_sub(x, j):
    nb = _R // (2 * j)
    x4 = x.reshape(nb, 2, j, _C)
    return jnp.concatenate([x4[:, 1:2], x4[:, 0:1]], axis=1).reshape(_R, _C)


def _xor_lane(x, j, bitj):
    s = j // _C
    left = jnp.concatenate([x[:, s:], x[:, :s]], axis=1)
    right = jnp.concatenate([x[:, -s:], x[:, :-s]], axis=1)
    return jnp.where(bitj, right, left)


def _stage_split(key, pay, j, sel_a):
    nb = _R // (2 * j)
    k4 = key.reshape(nb, 2, j, _C)
    p4 = pay.reshape(nb, 2, j, _C)
    ka, kb = k4[:, 0], k4[:, 1]
    pa, pb = p4[:, 0], p4[:, 1]
    mn = jnp.minimum(ka, kb)
    mx = jnp.maximum(ka, kb)
    if sel_a is None:
        na, nbv = mn, mx
    else:
        na = jnp.where(sel_a, mn, mx)
        nbv = jnp.where(sel_a, mx, mn)
    pa2 = jnp.where(na != ka, pb, pa)
    pb2 = jnp.where(nbv != kb, pa, pb)
    key = jnp.stack([na, nbv], axis=1).reshape(_R, _C)
    pay = jnp.stack([pa2, pb2], axis=1).reshape(_R, _C)
    return key, pay


def _bitonic_sort(key, pay, rb, cb):
    for lk in range(1, 15):
        for lj in range(lk - 1, -1, -1):
            j = 1 << lj
            if 3 <= lj <= 6:
                if lk == 14:
                    sel_a = None
                elif lk <= 6:
                    nb = _R // (2 * j)
                    sel_a = rb[lk].reshape(nb, 2, j, 1)[:, 0] == 0
                else:
                    sel_a = (cb[lk - 7] == 0).reshape(1, 1, _C)
                key, pay = _stage_split(key, pay, j, sel_a)
                continue
            bj = rb[lj] if lj <= 6 else cb[lj - 7]
            if lk == 14:
                sel = bj == 0
            else:
                bk = rb[lk] if lk <= 6 else cb[lk - 7]
                sel = bj == bk
            if lj <= 6:
                p_key = _xor_sub(key, j)
                p_pay = _xor_sub(pay, j)
            else:
                bjm = bj != 0
                p_key = _xor_lane(key, j, bjm)
                p_pay = _xor_lane(pay, j, bjm)
            new_key = jnp.where(sel, jnp.minimum(key, p_key),
                                jnp.maximum(key, p_key))
            swap = new_key != key
            pay = jnp.where(swap, p_pay, pay)
            key = new_key
    return key, pay


def _body(yp_ref, yt_ref, out_ref):
    rows = jax.lax.broadcasted_iota(jnp.int32, (_R, 1), 0)
    cols = jax.lax.broadcasted_iota(jnp.int32, (1, _C), 1)
    rb = [(rows >> b) & 1 for b in range(7)]
    cb = [(cols >> b) & 1 for b in range(7)]
    flat = (cols * _C + rows).astype(jnp.float32)

    _, z = _bitonic_sort(yp_ref[...], yt_ref[...], rb, cb)
    _, u = _bitonic_sort(z, jnp.broadcast_to(flat, (_R, _C)), rb, cb)

    num = jnp.sum((flat - _MEAN) * (u - _MEAN))
    loss = 1.0 - num / (jnp.float32(_SUMSQ) + 1e-8)
    out_ref[:, :] = jnp.full((8, _C), loss, jnp.float32)


def kernel(y_pred, y_true):
    yp = y_pred.reshape(_R, _C)
    yt = y_true.reshape(_R, _C)
    out = pl.pallas_call(
        _body,
        in_specs=[
            pl.BlockSpec((_R, _C), lambda: (0, 0)),
            pl.BlockSpec((_R, _C), lambda: (0, 0)),
        ],
        out_specs=pl.BlockSpec((8, _C), lambda: (0, 0)),
        out_shape=jax.ShapeDtypeStruct((8, _C), jnp.float32),
        grid=(),
    )(yp, yt)
    return out[0, 0]


# speedup vs baseline: 18.2887x; 1.0027x over previous
import jax
import jax.numpy as jnp
from jax.experimental import pallas as pl

_N = 16384
_R = 128
_C = 128
_MEAN = (_N - 1) / 2.0
_SUMSQ = float(_N) * (float(_N) ** 2 - 1.0) / 12.0


def _xor_sub(x, j):
    nb = _R // (2 * j)
    x4 = x.reshape(nb, 2, j, _C)
    return jnp.concatenate([x4[:, 1:2], x4[:, 0:1]], axis=1).reshape(_R, _C)


def _xor_lane(x, j, bitj):
    s = j // _C
    left = jnp.concatenate([x[:, s:], x[:, :s]], axis=1)
    right = jnp.concatenate([x[:, -s:], x[:, :-s]], axis=1)
    return jnp.where(bitj, right, left)


def _stage_split(key, pay, j, sel_a):
    nb = _R // (2 * j)
    k4 = key.reshape(nb, 2, j, _C)
    p4 = pay.reshape(nb, 2, j, _C)
    ka, kb = k4[:, 0], k4[:, 1]
    pa, pb = p4[:, 0], p4[:, 1]
    mn = jnp.minimum(ka, kb)
    mx = jnp.maximum(ka, kb)
    if sel_a is None:
        na, nbv = mn, mx
    else:
        na = jnp.where(sel_a, mn, mx)
        nbv = jnp.where(sel_a, mx, mn)
    pa2 = jnp.where(na != ka, pb, pa)
    pb2 = jnp.where(nbv != kb, pa, pb)
    key = jnp.stack([na, nbv], axis=1).reshape(_R, _C)
    pay = jnp.stack([pa2, pb2], axis=1).reshape(_R, _C)
    return key, pay


def _bitpat(b, rb, cb):
    if b <= 3:
        return "r", rb[b + 3]
    if b <= 10:
        return "c", cb[b - 4]
    return "r", rb[b - 11]


def _bitonic_sort(key, pay, rb, cb):
    for lk in range(1, 15):
        for lj in range(lk - 1, -1, -1):
            if lj <= 3:
                jr = 8 << lj
                nb = _R // (2 * jr)
                if lk == 14:
                    sel_a = None
                else:
                    kind, pat = _bitpat(lk, rb, cb)
                    if kind == "r":
                        sel_a = pat.reshape(nb, 2, jr, 1)[:, 0] == 0
                    else:
                        sel_a = (pat == 0).reshape(1, 1, _C)
                key, pay = _stage_split(key, pay, jr, sel_a)
                continue
            _, bj = _bitpat(lj, rb, cb)
            if lk == 14:
                sel = bj == 0
            else:
                _, bk = _bitpat(lk, rb, cb)
                sel = bj == bk
            if lj >= 11:
                jr = 1 << (lj - 11)
                p_key = _xor_sub(key, jr)
                p_pay = _xor_sub(pay, jr)
            else:
                jc = (1 << (lj - 4)) * _C
                bjm = bj != 0
                p_key = _xor_lane(key, jc, bjm)
                p_pay = _xor_lane(pay, jc, bjm)
            new_key = jnp.where(sel, jnp.minimum(key, p_key),
                                jnp.maximum(key, p_key))
            swap = new_key != key
            pay = jnp.where(swap, p_pay, pay)
            key = new_key
    return key, pay


def _body(yp_ref, yt_ref, out_ref):
    rows = jax.lax.broadcasted_iota(jnp.int32, (_R, 1), 0)
    cols = jax.lax.broadcasted_iota(jnp.int32, (1, _C), 1)
    rb = [(rows >> b) & 1 for b in range(7)]
    cb = [(cols >> b) & 1 for b in range(7)]
    flat = (((rows >> 3) & 15) | (cols << 4) | ((rows & 7) << 11)
            ).astype(jnp.float32)

    _, z = _bitonic_sort(yp_ref[...], yt_ref[...], rb, cb)
    _, u = _bitonic_sort(z, jnp.broadcast_to(flat, (_R, _C)), rb, cb)

    num = jnp.sum((flat - _MEAN) * (u - _MEAN))
    loss = 1.0 - num / (jnp.float32(_SUMSQ) + 1e-8)
    out_ref[:, :] = jnp.full((8, _C), loss, jnp.float32)


def kernel(y_pred, y_true):
    yp = y_pred.reshape(_R, _C)
    yt = y_true.reshape(_R, _C)
    out = pl.pallas_call(
        _body,
        in_specs=[
            pl.BlockSpec((_R, _C), lambda: (0, 0)),
            pl.BlockSpec((_R, _C), lambda: (0, 0)),
        ],
        out_specs=pl.BlockSpec((8, _C), lambda: (0, 0)),
        out_shape=jax.ShapeDtypeStruct((8, _C), jnp.float32),
        grid=(),
    )(yp, yt)
    return out[0, 0]


# speedup vs baseline: 23.5696x; 1.2888x over previous
import jax
import jax.numpy as jnp
from jax.experimental import pallas as pl
from jax.experimental.pallas import tpu as pltpu

_N = 16384
_R = 128
_C = 128
_MEAN = (_N - 1) / 2.0
_SUMSQ = float(_N) * (float(_N) ** 2 - 1.0) / 12.0


def _xor_sub(x, j):
    nb = _R // (2 * j)
    x4 = x.reshape(nb, 2, j, _C)
    return jnp.concatenate([x4[:, 1:2], x4[:, 0:1]], axis=1).reshape(_R, _C)


def _xor_lane(x, j, idx):
    return jnp.take_along_axis(x, idx, axis=1)


def _stage_split(key, pay, j, sel_a):
    nb = _R // (2 * j)
    k4 = key.reshape(nb, 2, j, _C)
    p4 = pay.reshape(nb, 2, j, _C)
    ka, kb = k4[:, 0], k4[:, 1]
    pa, pb = p4[:, 0], p4[:, 1]
    mn = jnp.minimum(ka, kb)
    mx = jnp.maximum(ka, kb)
    if sel_a is None:
        na, nbv = mn, mx
    else:
        na = jnp.where(sel_a, mn, mx)
        nbv = jnp.where(sel_a, mx, mn)
    pa2 = jnp.where(na != ka, pb, pa)
    pb2 = jnp.where(nbv != kb, pa, pb)
    key = jnp.stack([na, nbv], axis=1).reshape(_R, _C)
    pay = jnp.stack([pa2, pb2], axis=1).reshape(_R, _C)
    return key, pay


def _bitpat(b, rb, cb):
    if b <= 3:
        return "r", rb[b + 3]
    if b <= 10:
        return "c", cb[b - 4]
    return "r", rb[b - 11]


def _bitonic_sort(key, pay, rb, cb):
    for lk in range(1, 15):
        for lj in range(lk - 1, -1, -1):
            if lj <= 3:
                jr = 8 << lj
                nb = _R // (2 * jr)
                if lk == 14:
                    sel_a = None
                else:
                    kind, pat = _bitpat(lk, rb, cb)
                    if kind == "r":
                        sel_a = pat.reshape(nb, 2, jr, 1)[:, 0] == 0
                    else:
                        sel_a = (pat == 0).reshape(1, 1, _C)
                key, pay = _stage_split(key, pay, jr, sel_a)
                continue
            _, bj = _bitpat(lj, rb, cb)
            if lk == 14:
                sel = bj == 0
            else:
                _, bk = _bitpat(lk, rb, cb)
                sel = bj == bk
            if lj >= 11:
                jr = 1 << (lj - 11)
                p_key = _xor_sub(key, jr)
                p_pay = _xor_sub(pay, jr)
            else:
                s = 1 << (lj - 4)
                idx = jnp.broadcast_to(
                    jax.lax.broadcasted_iota(jnp.int32, (1, _C), 1) ^ s,
                    (_R, _C))
                p_key = _xor_lane(key, s, idx)
                p_pay = _xor_lane(pay, s, idx)
            new_key = jnp.where(sel, jnp.minimum(key, p_key),
                                jnp.maximum(key, p_key))
            swap = new_key != key
            pay = jnp.where(swap, p_pay, pay)
            key = new_key
    return key, pay


def _body(yp_ref, yt_ref, out_ref):
    rows = jax.lax.broadcasted_iota(jnp.int32, (_R, 1), 0)
    cols = jax.lax.broadcasted_iota(jnp.int32, (1, _C), 1)
    rb = [(rows >> b) & 1 for b in range(7)]
    cb = [(cols >> b) & 1 for b in range(7)]
    flat = (((rows >> 3) & 15) | (cols << 4) | ((rows & 7) << 11)
            ).astype(jnp.float32)

    _, z = _bitonic_sort(yp_ref[...], yt_ref[...], rb, cb)
    _, u = _bitonic_sort(z, jnp.broadcast_to(flat, (_R, _C)), rb, cb)

    num = jnp.sum((flat - _MEAN) * (u - _MEAN))
    loss = 1.0 - num / (jnp.float32(_SUMSQ) + 1e-8)
    out_ref[0, 0] = loss


def kernel(y_pred, y_true):
    yp = y_pred.reshape(_R, _C)
    yt = y_true.reshape(_R, _C)
    out = pl.pallas_call(
        _body,
        in_specs=[
            pl.BlockSpec((_R, _C), lambda: (0, 0)),
            pl.BlockSpec((_R, _C), lambda: (0, 0)),
        ],
        out_specs=pl.BlockSpec(memory_space=pltpu.SMEM),
        out_shape=jax.ShapeDtypeStruct((1, 1), jnp.float32),
        grid=(),
    )(yp, yt)
    return out.reshape(())


# speedup vs baseline: 24.1443x; 1.0244x over previous
import jax
import jax.numpy as jnp
from jax.experimental import pallas as pl
from jax.experimental.pallas import tpu as pltpu

_N = 16384
_R = 128
_C = 128
_MEAN = (_N - 1) / 2.0
_SUMSQ = float(_N) * (float(_N) ** 2 - 1.0) / 12.0


def _xor_sub(x, j):
    nb = _R // (2 * j)
    x4 = x.reshape(nb, 2, j, _C)
    return jnp.concatenate([x4[:, 1:2], x4[:, 0:1]], axis=1).reshape(_R, _C)


def _xor_lane(x, j, idx):
    return jnp.take_along_axis(x, idx, axis=1)


def _stage_split(key, pay, j, sel_a):
    nb = _R // (2 * j)
    k4 = key.reshape(nb, 2, j, _C)
    p4 = pay.reshape(nb, 2, j, _C)
    ka, kb = k4[:, 0], k4[:, 1]
    pa, pb = p4[:, 0], p4[:, 1]
    mn = jnp.minimum(ka, kb)
    mx = jnp.maximum(ka, kb)
    if sel_a is None:
        na, nbv = mn, mx
    else:
        na = jnp.where(sel_a, mn, mx)
        nbv = jnp.where(sel_a, mx, mn)
    pa2 = jnp.where(na != ka, pb, pa)
    pb2 = jnp.where(nbv != kb, pa, pb)
    key = jnp.stack([na, nbv], axis=1).reshape(_R, _C)
    pay = jnp.stack([pa2, pb2], axis=1).reshape(_R, _C)
    return key, pay


def _bitpat(b, rb, cb):
    if b <= 3:
        return "r", rb[b + 3]
    if b <= 6:
        return "r", rb[b - 4]
    return "c", cb[b - 7]


def _bitonic_sort(key, pay, rb, cb):
    for lk in range(1, 15):
        for lj in range(lk - 1, -1, -1):
            if lj <= 3:
                jr = 8 << lj
                nb = _R // (2 * jr)
                if lk == 14:
                    sel_a = None
                else:
                    kind, pat = _bitpat(lk, rb, cb)
                    if kind == "r":
                        sel_a = pat.reshape(nb, 2, jr, 1)[:, 0] == 0
                    else:
                        sel_a = (pat == 0).reshape(1, 1, _C)
                key, pay = _stage_split(key, pay, jr, sel_a)
                continue
            _, bj = _bitpat(lj, rb, cb)
            if lk == 14:
                sel = bj == 0
            else:
                _, bk = _bitpat(lk, rb, cb)
                sel = bj == bk
            if lj <= 6:
                jr = 1 << (lj - 4)
                p_key = _xor_sub(key, jr)
                p_pay = _xor_sub(pay, jr)
            else:
                s = 1 << (lj - 7)
                idx = jnp.broadcast_to(
                    jax.lax.broadcasted_iota(jnp.int32, (1, _C), 1) ^ s,
                    (_R, _C))
                p_key = _xor_lane(key, s, idx)
                p_pay = _xor_lane(pay, s, idx)
            new_key = jnp.where(sel, jnp.minimum(key, p_key),
                                jnp.maximum(key, p_key))
            swap = new_key != key
            pay = jnp.where(swap, p_pay, pay)
            key = new_key
    return key, pay


def _body(yp_ref, yt_ref, out_ref):
    rows = jax.lax.broadcasted_iota(jnp.int32, (_R, 1), 0)
    cols = jax.lax.broadcasted_iota(jnp.int32, (1, _C), 1)
    rb = [(rows >> b) & 1 for b in range(7)]
    cb = [(cols >> b) & 1 for b in range(7)]
    flat = (((rows >> 3) & 15) | ((rows & 7) << 4) | (cols << 7)
            ).astype(jnp.float32)

    _, z = _bitonic_sort(yp_ref[...], yt_ref[...], rb, cb)
    _, u = _bitonic_sort(z, jnp.broadcast_to(flat, (_R, _C)), rb, cb)

    num = jnp.sum((flat - _MEAN) * (u - _MEAN))
    loss = 1.0 - num / (jnp.float32(_SUMSQ) + 1e-8)
    out_ref[0, 0] = loss


def kernel(y_pred, y_true):
    yp = y_pred.reshape(_R, _C)
    yt = y_true.reshape(_R, _C)
    out = pl.pallas_call(
        _body,
        in_specs=[
            pl.BlockSpec((_R, _C), lambda: (0, 0)),
            pl.BlockSpec((_R, _C), lambda: (0, 0)),
        ],
        out_specs=pl.BlockSpec(memory_space=pltpu.SMEM),
        out_shape=jax.ShapeDtypeStruct((1, 1), jnp.float32),
        grid=(),
    )(yp, yt)
    return out.reshape(())


# speedup vs baseline: 25.1843x; 1.0431x over previous
import jax
import jax.numpy as jnp
from jax.experimental import pallas as pl
from jax.experimental.pallas import tpu as pltpu

_N = 16384
_R = 128
_C = 128
_MEAN = (_N - 1) / 2.0
_SUMSQ = float(_N) * (float(_N) ** 2 - 1.0) / 12.0


def _xor_sub(x, j):
    nb = _R // (2 * j)
    x4 = x.reshape(nb, 2, j, _C)
    return jnp.concatenate([x4[:, 1:2], x4[:, 0:1]], axis=1).reshape(_R, _C)


def _xor_lane(x, j, idx):
    return jnp.take_along_axis(x, idx, axis=1)


def _stage_split(key, pay, j, sel_a):
    nb = _R // (2 * j)
    k4 = key.reshape(nb, 2, j, _C)
    p4 = pay.reshape(nb, 2, j, _C)
    ka, kb = k4[:, 0], k4[:, 1]
    pa, pb = p4[:, 0], p4[:, 1]
    mn = jnp.minimum(ka, kb)
    mx = jnp.maximum(ka, kb)
    if sel_a is None:
        na, nbv = mn, mx
    else:
        na = jnp.where(sel_a, mn, mx)
        nbv = jnp.where(sel_a, mx, mn)
    pa2 = jnp.where(na != ka, pb, pa)
    pb2 = jnp.where(nbv != kb, pa, pb)
    key = jnp.stack([na, nbv], axis=1).reshape(_R, _C)
    pay = jnp.stack([pa2, pb2], axis=1).reshape(_R, _C)
    return key, pay


def _bitpat(b, rb, cb):
    if b <= 3:
        return "r", rb[b + 3]
    if b <= 6:
        return "c", cb[b - 4]
    if b <= 9:
        return "r", rb[b - 7]
    return "c", cb[b - 7]


def _bitonic_sort(key, pay, rb, cb):
    for lk in range(1, 15):
        for lj in range(lk - 1, -1, -1):
            if lj <= 3:
                jr = 8 << lj
                nb = _R // (2 * jr)
                if lk == 14:
                    sel_a = None
                else:
                    kind, pat = _bitpat(lk, rb, cb)
                    if kind == "r":
                        sel_a = pat.reshape(nb, 2, jr, 1)[:, 0] == 0
                    else:
                        sel_a = (pat == 0).reshape(1, 1, _C)
                key, pay = _stage_split(key, pay, jr, sel_a)
                continue
            _, bj = _bitpat(lj, rb, cb)
            if lk == 14:
                sel = bj == 0
            else:
                _, bk = _bitpat(lk, rb, cb)
                sel = bj == bk
            if 7 <= lj <= 9:
                jr = 1 << (lj - 7)
                p_key = _xor_sub(key, jr)
                p_pay = _xor_sub(pay, jr)
            else:
                s = (1 << (lj - 4)) if lj <= 6 else (1 << (lj - 7))
                idx = jnp.broadcast_to(
                    jax.lax.broadcasted_iota(jnp.int32, (1, _C), 1) ^ s,
                    (_R, _C))
                p_key = _xor_lane(key, s, idx)
                p_pay = _xor_lane(pay, s, idx)
            new_key = jnp.where(sel, jnp.minimum(key, p_key),
                                jnp.maximum(key, p_key))
            swap = new_key != key
            pay = jnp.where(swap, p_pay, pay)
            key = new_key
    return key, pay


def _body(yp_ref, yt_ref, out_ref):
    rows = jax.lax.broadcasted_iota(jnp.int32, (_R, 1), 0)
    cols = jax.lax.broadcasted_iota(jnp.int32, (1, _C), 1)
    rb = [(rows >> b) & 1 for b in range(7)]
    cb = [(cols >> b) & 1 for b in range(7)]
    flat = (((rows >> 3) & 15) | ((cols & 7) << 4) | ((rows & 7) << 7)
            | ((cols >> 3) << 10)).astype(jnp.float32)

    _, z = _bitonic_sort(yp_ref[...], yt_ref[...], rb, cb)
    _, u = _bitonic_sort(z, jnp.broadcast_to(flat, (_R, _C)), rb, cb)

    num = jnp.sum((flat - _MEAN) * (u - _MEAN))
    loss = 1.0 - num / (jnp.float32(_SUMSQ) + 1e-8)
    out_ref[0, 0] = loss


def kernel(y_pred, y_true):
    yp = y_pred.reshape(_R, _C)
    yt = y_true.reshape(_R, _C)
    out = pl.pallas_call(
        _body,
        in_specs=[
            pl.BlockSpec((_R, _C), lambda: (0, 0)),
            pl.BlockSpec((_R, _C), lambda: (0, 0)),
        ],
        out_specs=pl.BlockSpec(memory_space=pltpu.SMEM),
        out_shape=jax.ShapeDtypeStruct((1, 1), jnp.float32),
        grid=(),
    )(yp, yt)
    return out.reshape(())
